# baseline (speedup 1.0000x reference)
import functools

import jax
import jax.numpy as jnp
from jax import lax
from jax.experimental import pallas as pl
from jax.experimental.pallas import tpu as pltpu
from jax.experimental.pallas import tpu_sc as plsc

N = 100000
F = 10
V = 500
D = 128

NC = 2
NS = 16
L = 16

DSLICES = 8
GROUPS = 4
NP = 100352
ROWS_PER_GROUP = NP // GROUPS
CHUNK = 512
CHUNKS = ROWS_PER_GROUP // CHUNK
BLOCKS = CHUNK // L
NBUF = 2


def _sc_body(xt_hbm, em2_hbm, out_hbm, tbl_v, xt_v, out_v, in_sems, out_sems):
    cid = lax.axis_index("c")
    sid = lax.axis_index("s")
    wid = sid * NC + cid
    k = lax.rem(wid, DSLICES)
    g = wid // DSLICES

    pltpu.sync_copy(em2_hbm.at[k], tbl_v)
    row0 = g * ROWS_PER_GROUP

    def start_in(c, buf):
        base = row0 + c * CHUNK
        pltpu.make_async_copy(
            xt_hbm.at[:, pl.ds(base, CHUNK)], xt_v.at[buf], in_sems.at[buf]
        ).start()

    def wait_in(buf):
        pltpu.make_async_copy(
            xt_hbm.at[:, pl.ds(0, CHUNK)], xt_v.at[buf], in_sems.at[buf]
        ).wait()

    def start_out(c, buf):
        base = row0 + c * CHUNK
        pltpu.make_async_copy(
            out_v.at[buf], out_hbm.at[k, pl.ds(base, CHUNK)],
            out_sems.at[buf],
        ).start()

    def wait_out(buf):
        pltpu.make_async_copy(
            out_v.at[buf], out_hbm.at[0, pl.ds(0, CHUNK)],
            out_sems.at[buf],
        ).wait()

    start_in(0, 0)

    def chunk_body(c, carry):
        buf = lax.rem(c, NBUF)
        wait_in(buf)

        @pl.when(c + 1 < CHUNKS)
        def _():
            start_in(c + 1, lax.rem(c + 1, NBUF))

        def block_body(b, bcarry):
            b16 = pl.multiple_of(b * L, L)
            rows = b * L + lax.iota(jnp.int32, L)
            idx = [xt_v[buf, i, pl.ds(b16, L)] for i in range(F)]
            for d in range(L):
                dcol = jnp.full((L,), d, jnp.int32)
                gs = [plsc.load_gather(tbl_v, [idx[i], dcol]) for i in range(F)]
                while len(gs) > 1:
                    gs = [gs[j] + gs[j + 1] for j in range(0, len(gs) - 1, 2)] + (
                        [gs[-1]] if len(gs) % 2 else [])
                plsc.store_scatter(out_v.at[buf], [rows, dcol], gs[0])
            return bcarry

        @pl.when(c >= NBUF)
        def _():
            wait_out(buf)

        lax.fori_loop(0, BLOCKS, block_body, 0)
        start_out(c, buf)
        return carry

    lax.fori_loop(0, CHUNKS, chunk_body, 0)
    wait_out(lax.rem(CHUNKS - 2, NBUF))
    wait_out(lax.rem(CHUNKS - 1, NBUF))


def kernel(x, emb):
    if x.ndim == 1:
        x = x[:, None]
    xt = x.astype(jnp.int32).T + (V * jnp.arange(F, dtype=jnp.int32))[:, None]
    xt = jnp.pad(xt, ((0, 0), (0, NP - N)))
    em2 = emb.reshape(F * V, DSLICES, L).transpose(1, 0, 2)

    mesh = plsc.VectorSubcoreMesh(
        core_axis_name="c", subcore_axis_name="s", num_cores=NC,
        num_subcores=NS)
    run = pl.kernel(
        _sc_body,
        out_type=jax.ShapeDtypeStruct((DSLICES, NP, L), jnp.float32),
        mesh=mesh,
        compiler_params=pltpu.CompilerParams(
            use_tc_tiling_on_sc=False, needs_layout_passes=False),
        scratch_types=[
            pltpu.VMEM((F * V, L), jnp.float32),
            pltpu.VMEM((NBUF, F, CHUNK), jnp.int32),
            pltpu.VMEM((NBUF, CHUNK, L), jnp.float32),
            pltpu.SemaphoreType.DMA((NBUF,)),
            pltpu.SemaphoreType.DMA((NBUF,)),
        ],
    )
    out3 = run(xt, em2)
    return out3.transpose(1, 0, 2).reshape(NP, D)[:N]
